# Initial kernel scaffold; baseline (speedup 1.0000x reference)
import jax
import jax.numpy as jnp
from jax.experimental import pallas as pl


def kernel(node_feats, edge_feats, edge_index, W_node, b_node, W_edge, b_edge, W1, b1, W2, b2, gamma, beta):
    raise NotImplementedError("write your pallas kernel here")


# speedup vs baseline: 1.9835x; 1.9835x over previous
import functools

import jax
import jax.numpy as jnp
from jax import lax
from jax.experimental import pallas as pl
from jax.experimental.pallas import tpu as pltpu
from jax.experimental.pallas import tpu_sc as plsc

N = 10000
E = 320000
D_IN = 128
D_EDGE = 16
H = 128
L = 3

NUM_CORES = 2
NUM_SUBCORES = 16
EPT = E // NUM_SUBCORES
CH = 80
NCH = EPT // CH
NPC = 5120
NPD = 5248
RPT = NPD // NUM_SUBCORES
OPT = NPC // NUM_SUBCORES
DUMP = NPC
ZB = 8
LANES = 16


def _sc_agg_body(h_hbm, e_hbm, src_hbm, dst_hbm, out_hbm,
                 dsti, srci, buf, idxt, zbuf, aggs, gsem):
    c = lax.axis_index("c")
    s = lax.axis_index("s")
    lo = c * NPC

    pltpu.sync_copy(dst_hbm.at[s], dsti)
    pltpu.sync_copy(src_hbm.at[s], srci)

    zv = jnp.zeros((LANES,), jnp.float32)
    for r in range(ZB):
        for q in range(H // LANES):
            zbuf[r, pl.ds(q * LANES, LANES)] = zv

    def _zcopy(k, _):
        pltpu.sync_copy(zbuf, aggs.at[pl.ds(s * RPT + k * ZB, ZB)])
        return 0

    lax.fori_loop(0, RPT // ZB, _zcopy, 0)
    plsc.subcore_barrier()

    def _chunk(j, _):
        base = s * EPT + j * CH
        pltpu.sync_copy(e_hbm.at[pl.ds(base, CH)], buf)
        pltpu.async_copy(h_hbm.at[dsti.at[j]], buf, gsem, add=True).wait()

        for q in range(CH // LANES):
            sl = pl.ds(q * LANES, LANES)
            u = srci[j, sl] - lo
            valid = (u >= 0) & (u < NPC)
            idxt[0, sl] = jnp.where(valid, u, DUMP)

        def _relu_row(r, _):
            for q in range(H // LANES):
                sl = pl.ds(q * LANES, LANES)
                buf[r, sl] = jnp.maximum(buf[r, sl], 0.0)
            return 0

        lax.fori_loop(0, CH, _relu_row, 0)
        pltpu.sync_copy(buf, aggs.at[idxt.at[0]], add=True)
        return 0

    lax.fori_loop(0, NCH, _chunk, 0)
    plsc.subcore_barrier()

    pltpu.sync_copy(aggs.at[pl.ds(s * OPT, OPT)],
                    out_hbm.at[c, pl.ds(s * OPT, OPT)])


@functools.cache
def _sc_agg():
    return pl.kernel(
        _sc_agg_body,
        out_type=jax.ShapeDtypeStruct((NUM_CORES, NPC, H), jnp.float32),
        mesh=plsc.VectorSubcoreMesh(
            core_axis_name="c", subcore_axis_name="s",
            num_cores=NUM_CORES, num_subcores=NUM_SUBCORES,
        ),
        scratch_types=[
            pltpu.VMEM((NCH, CH), jnp.int32),
            pltpu.VMEM((NCH, CH), jnp.int32),
            pltpu.VMEM((CH, H), jnp.float32),
            pltpu.VMEM((8, CH), jnp.int32),
            pltpu.VMEM((ZB, H), jnp.float32),
            pltpu.VMEM_SHARED((NPD, H), jnp.float32),
            pltpu.SemaphoreType.DMA,
        ],
    )


def _proj_body(x_ref, w_ref, b_ref, o_ref):
    o_ref[...] = (
        jnp.dot(x_ref[...], w_ref[...], preferred_element_type=jnp.float32)
        + b_ref[...]
    )


def _proj(x, w, b, block_rows):
    rows, d_in = x.shape
    grid = rows // block_rows
    return pl.pallas_call(
        _proj_body,
        grid=(grid,),
        in_specs=[
            pl.BlockSpec((block_rows, d_in), lambda i: (i, 0)),
            pl.BlockSpec((d_in, H), lambda i: (0, 0)),
            pl.BlockSpec((1, H), lambda i: (0, 0)),
        ],
        out_specs=pl.BlockSpec((block_rows, H), lambda i: (i, 0)),
        out_shape=jax.ShapeDtypeStruct((rows, H), jnp.float32),
    )(x, w, b.reshape(1, H))


def _mlp_body(h_ref, a_ref, w1_ref, b1_ref, w2_ref, b2_ref, g_ref, be_ref,
              o_ref):
    h = h_ref[...]
    new = h + a_ref[...]
    hid = jax.nn.gelu(
        jnp.dot(new, w1_ref[...], preferred_element_type=jnp.float32)
        + b1_ref[...]
    )
    new = (
        jnp.dot(hid, w2_ref[...], preferred_element_type=jnp.float32)
        + b2_ref[...]
    )
    x = new + h
    mu = jnp.mean(x, axis=-1, keepdims=True)
    var = jnp.mean((x - mu) ** 2, axis=-1, keepdims=True)
    o_ref[...] = (x - mu) / jnp.sqrt(var + 1e-5) * g_ref[...] + be_ref[...]


def _mlp(h, agg, w1, b1, w2, b2, g, be, block_rows=1000):
    grid = N // block_rows
    return pl.pallas_call(
        _mlp_body,
        grid=(grid,),
        in_specs=[
            pl.BlockSpec((block_rows, H), lambda i: (i, 0)),
            pl.BlockSpec((block_rows, H), lambda i: (i, 0)),
            pl.BlockSpec((H, H // 2), lambda i: (0, 0)),
            pl.BlockSpec((1, H // 2), lambda i: (0, 0)),
            pl.BlockSpec((H // 2, H), lambda i: (0, 0)),
            pl.BlockSpec((1, H), lambda i: (0, 0)),
            pl.BlockSpec((1, H), lambda i: (0, 0)),
            pl.BlockSpec((1, H), lambda i: (0, 0)),
        ],
        out_specs=pl.BlockSpec((block_rows, H), lambda i: (i, 0)),
        out_shape=jax.ShapeDtypeStruct((N, H), jnp.float32),
    )(h, agg, w1, b1.reshape(1, H // 2), w2, b2.reshape(1, H),
      g.reshape(1, H), be.reshape(1, H))


def kernel(node_feats, edge_feats, edge_index, W_node, b_node, W_edge, b_edge,
           W1, b1, W2, b2, gamma, beta):
    src = edge_index[0].astype(jnp.int32).reshape(NUM_SUBCORES, NCH, CH)
    dst = edge_index[1].astype(jnp.int32).reshape(NUM_SUBCORES, NCH, CH)

    h = _proj(node_feats, W_node, b_node, block_rows=1000)
    e = _proj(edge_feats, W_edge, b_edge, block_rows=2000)

    def layer(h, wts):
        w1, bb1, w2, bb2, g, be = wts
        agg2 = _sc_agg()(h, e, src, dst)
        agg = agg2.reshape(NUM_CORES * NPC, H)
        h = _mlp(h, agg, w1, bb1, w2, bb2, g, be)
        return h, None

    h, _ = lax.scan(layer, h, (W1, b1, W2, b2, gamma, beta))
    return h


# speedup vs baseline: 3.2902x; 1.6587x over previous
import functools

import jax
import jax.numpy as jnp
from jax import lax
from jax.experimental import pallas as pl
from jax.experimental.pallas import tpu as pltpu
from jax.experimental.pallas import tpu_sc as plsc

N = 10000
E = 320000
D_IN = 128
D_EDGE = 16
H = 128
L = 3

NUM_CORES = 2
NUM_SUBCORES = 16
EPT = E // NUM_SUBCORES
CH = 80
NCH = EPT // CH
NPC = 5120
NPD = 5248
RPT = NPD // NUM_SUBCORES
OPT = NPC // NUM_SUBCORES
DUMP = NPC
ZB = 8
NBUF = 5
GRP = 25
GRP_OUT = GRP // NBUF
NGRP = NCH // GRP
LANES = 16


def _sc_agg_body(h_hbm, e_hbm, src_hbm, dst_hbm, out_hbm,
                 dsti, srci, b0, b1, b2, b3, b4, idxt, zbuf, aggs, *sems):
    bufs = (b0, b1, b2, b3, b4)
    esem = sems[0:NBUF]
    gsem = sems[NBUF:2 * NBUF]
    ssem = sems[2 * NBUF:3 * NBUF]
    c = lax.axis_index("c")
    s = lax.axis_index("s")
    lo = c * NPC

    zv = jnp.zeros((LANES,), jnp.float32)
    for r in range(ZB):
        for q in range(H // LANES):
            zbuf[r, pl.ds(q * LANES, LANES)] = zv

    def _zcopy(k, _):
        pltpu.sync_copy(zbuf, aggs.at[pl.ds(s * RPT + k * ZB, ZB)])
        return 0

    lax.fori_loop(0, RPT // ZB, _zcopy, 0)
    plsc.subcore_barrier()

    def _outer(k, _):
        @pl.when(lax.rem(k, GRP_OUT) == 0)
        def _():
            g = lax.div(k, GRP_OUT)
            pltpu.sync_copy(dst_hbm.at[s, g], dsti)
            pltpu.sync_copy(src_hbm.at[s, g], srci)

        for b in range(NBUF):
            base = s * EPT + (k * NBUF + b) * CH

            @pl.when(k > 0)
            def _(b=b):
                pltpu.make_async_copy(
                    bufs[b], aggs.at[idxt.at[b]], ssem[b]).wait()

            pltpu.async_copy(e_hbm.at[pl.ds(base, CH)], bufs[b], esem[b])

        for b in range(NBUF):
            jj = lax.rem(k, GRP_OUT) * NBUF + b
            base = s * EPT + (k * NBUF + b) * CH
            pltpu.make_async_copy(
                e_hbm.at[pl.ds(base, CH)], bufs[b], esem[b]).wait()
            pltpu.async_copy(h_hbm.at[dsti.at[jj]], bufs[b], gsem[b],
                             add=True)

        for b in range(NBUF):
            jj = lax.rem(k, GRP_OUT) * NBUF + b
            pltpu.make_async_copy(
                h_hbm.at[dsti.at[jj]], bufs[b], gsem[b]).wait()

            for q in range(CH // LANES):
                sl = pl.ds(q * LANES, LANES)
                u = srci[jj, sl] - lo
                valid = (u >= 0) & (u < NPC)
                idxt[b, sl] = jnp.where(valid, u, DUMP)

            def _relu_row(r, _, b=b):
                for q in range(H // LANES):
                    sl = pl.ds(q * LANES, LANES)
                    bufs[b][r, sl] = jnp.maximum(bufs[b][r, sl], 0.0)
                return 0

            lax.fori_loop(0, CH, _relu_row, 0)
            pltpu.async_copy(bufs[b], aggs.at[idxt.at[b]], ssem[b], add=True)
        return 0

    lax.fori_loop(0, NCH // NBUF, _outer, 0)
    for b in range(NBUF):
        pltpu.make_async_copy(bufs[b], aggs.at[idxt.at[b]], ssem[b]).wait()
    plsc.subcore_barrier()

    pltpu.sync_copy(aggs.at[pl.ds(s * OPT, OPT)],
                    out_hbm.at[c, pl.ds(s * OPT, OPT)])


@functools.cache
def _sc_agg():
    return pl.kernel(
        _sc_agg_body,
        out_type=jax.ShapeDtypeStruct((NUM_CORES, NPC, H), jnp.float32),
        mesh=plsc.VectorSubcoreMesh(
            core_axis_name="c", subcore_axis_name="s",
            num_cores=NUM_CORES, num_subcores=NUM_SUBCORES,
        ),
        scratch_types=[
            pltpu.VMEM((GRP, CH), jnp.int32),
            pltpu.VMEM((GRP, CH), jnp.int32),
        ] + [pltpu.VMEM((CH, H), jnp.float32) for _ in range(NBUF)] + [
            pltpu.VMEM((8, CH), jnp.int32),
            pltpu.VMEM((ZB, H), jnp.float32),
            pltpu.VMEM_SHARED((NPD, H), jnp.float32),
        ] + [pltpu.SemaphoreType.DMA for _ in range(3 * NBUF)],
    )


def _proj_body(x_ref, w_ref, b_ref, o_ref):
    o_ref[...] = (
        jnp.dot(x_ref[...], w_ref[...], preferred_element_type=jnp.float32)
        + b_ref[...]
    )


def _proj(x, w, b, block_rows):
    rows, d_in = x.shape
    grid = rows // block_rows
    return pl.pallas_call(
        _proj_body,
        grid=(grid,),
        in_specs=[
            pl.BlockSpec((block_rows, d_in), lambda i: (i, 0)),
            pl.BlockSpec((d_in, H), lambda i: (0, 0)),
            pl.BlockSpec((1, H), lambda i: (0, 0)),
        ],
        out_specs=pl.BlockSpec((block_rows, H), lambda i: (i, 0)),
        out_shape=jax.ShapeDtypeStruct((rows, H), jnp.float32),
    )(x, w, b.reshape(1, H))


def _mlp_body(h_ref, a_ref, w1_ref, b1_ref, w2_ref, b2_ref, g_ref, be_ref,
              o_ref):
    h = h_ref[...]
    new = h + a_ref[...]
    hid = jax.nn.gelu(
        jnp.dot(new, w1_ref[...], preferred_element_type=jnp.float32)
        + b1_ref[...]
    )
    new = (
        jnp.dot(hid, w2_ref[...], preferred_element_type=jnp.float32)
        + b2_ref[...]
    )
    x = new + h
    mu = jnp.mean(x, axis=-1, keepdims=True)
    var = jnp.mean((x - mu) ** 2, axis=-1, keepdims=True)
    o_ref[...] = (x - mu) / jnp.sqrt(var + 1e-5) * g_ref[...] + be_ref[...]


def _mlp(h, agg, w1, b1, w2, b2, g, be, block_rows=1000):
    grid = N // block_rows
    return pl.pallas_call(
        _mlp_body,
        grid=(grid,),
        in_specs=[
            pl.BlockSpec((block_rows, H), lambda i: (i, 0)),
            pl.BlockSpec((block_rows, H), lambda i: (i, 0)),
            pl.BlockSpec((H, H // 2), lambda i: (0, 0)),
            pl.BlockSpec((1, H // 2), lambda i: (0, 0)),
            pl.BlockSpec((H // 2, H), lambda i: (0, 0)),
            pl.BlockSpec((1, H), lambda i: (0, 0)),
            pl.BlockSpec((1, H), lambda i: (0, 0)),
            pl.BlockSpec((1, H), lambda i: (0, 0)),
        ],
        out_specs=pl.BlockSpec((block_rows, H), lambda i: (i, 0)),
        out_shape=jax.ShapeDtypeStruct((N, H), jnp.float32),
    )(h, agg, w1, b1.reshape(1, H // 2), w2, b2.reshape(1, H),
      g.reshape(1, H), be.reshape(1, H))


def kernel(node_feats, edge_feats, edge_index, W_node, b_node, W_edge, b_edge,
           W1, b1, W2, b2, gamma, beta):
    src = edge_index[0].astype(jnp.int32).reshape(NUM_SUBCORES, NGRP, GRP, CH)
    dst = edge_index[1].astype(jnp.int32).reshape(NUM_SUBCORES, NGRP, GRP, CH)

    h = _proj(node_feats, W_node, b_node, block_rows=1000)
    e = _proj(edge_feats, W_edge, b_edge, block_rows=2000)

    def layer(h, wts):
        w1, bb1, w2, bb2, g, be = wts
        agg2 = _sc_agg()(h, e, src, dst)
        agg = agg2.reshape(NUM_CORES * NPC, H)
        h = _mlp(h, agg, w1, bb1, w2, bb2, g, be)
        return h, None

    h, _ = lax.scan(layer, h, (W1, b1, W2, b2, gamma, beta))
    return h
